# Initial kernel scaffold; baseline (speedup 1.0000x reference)
import jax
import jax.numpy as jnp
from jax.experimental import pallas as pl


def kernel(x, embed_weight, fc_weight, fc_bias):
    raise NotImplementedError("write your pallas kernel here")


# speedup vs baseline: 103.6032x; 103.6032x over previous
import functools

import jax
import jax.numpy as jnp
from jax import lax
from jax.experimental import pallas as pl
from jax.experimental.pallas import tpu as pltpu
from jax.experimental.pallas import tpu_sc as plsc

B = 16384
L = 200
V = 17
D = 16
VP = 32
CNT = L * VP
NC, NS = 2, 16
NW = NC * NS
CHUNK = (B * L) // NW
BLK = 400
NBLK = CHUNK // BLK

_mesh = plsc.VectorSubcoreMesh(core_axis_name="c", subcore_axis_name="s")


@functools.partial(
    pl.kernel,
    out_type=jax.ShapeDtypeStruct((NW, CNT), jnp.float32),
    mesh=_mesh,
    compiler_params=pltpu.CompilerParams(needs_layout_passes=False),
    scratch_types=[
        pltpu.VMEM((CHUNK,), jnp.int32),
        pltpu.VMEM((CNT,), jnp.float32),
        pltpu.VMEM((BLK,), jnp.int32),
    ],
)
def _sc_hist(x_hbm, out_hbm, xbuf, cnt, lmul):
    wid = lax.axis_index("s") * NC + lax.axis_index("c")

    def zbody(j, _):
        cnt[pl.ds(j * 16, 16)] = jnp.zeros((16,), jnp.float32)
        return 0
    lax.fori_loop(0, CNT // 16, zbody, 0)

    for s in range(BLK // 16):
        t = lax.iota(jnp.int32, 16) + s * 16
        l = jnp.where(t >= L, t - L, t)
        lmul[pl.ds(s * 16, 16)] = l * VP

    pltpu.sync_copy(x_hbm.at[pl.ds(wid * CHUNK, CHUNK)], xbuf)

    ones = jnp.ones((16,), jnp.float32)

    def mbody(blk, _):
        base = blk * BLK
        for s in range(BLK // 16):
            v = xbuf[pl.ds(base + s * 16, 16)]
            idx = lmul[pl.ds(s * 16, 16)] + v
            plsc.addupdate_scatter(cnt, [idx], ones)
        return 0
    lax.fori_loop(0, NBLK, mbody, 0)

    pltpu.sync_copy(cnt, out_hbm.at[wid])


def _tc_body(cnt_ref, embed_ref, fcw_ref, bias_ref, out_ref):
    c = jnp.sum(cnt_ref[...], axis=0)
    m = jnp.dot(c, embed_ref[...],
                preferred_element_type=jnp.float32) * (1.0 / B)
    out = lax.dot_general(m, fcw_ref[...],
                          (((1,), (1,)), ((), ())),
                          preferred_element_type=jnp.float32)
    out_ref[...] = out + bias_ref[...]


def kernel(x, embed_weight, fc_weight, fc_bias):
    xflat = x.astype(jnp.int32).reshape(-1)
    counts = _sc_hist(xflat)
    cnt3 = counts.reshape(NW, L, VP)
    embed_p = jnp.zeros((VP, D), jnp.float32).at[:V].set(embed_weight)
    out = pl.pallas_call(
        _tc_body,
        out_shape=jax.ShapeDtypeStruct((L, V), jnp.float32),
    )(cnt3, embed_p, fc_weight, fc_bias.reshape(1, V))
    return out


# speedup vs baseline: 112.1760x; 1.0827x over previous
import functools

import jax
import jax.numpy as jnp
from jax import lax
from jax.experimental import pallas as pl
from jax.experimental.pallas import tpu as pltpu
from jax.experimental.pallas import tpu_sc as plsc

B = 16384
L = 200
V = 17
D = 16
LP = 256
CNT = V * LP
NC, NS = 2, 16
NW = NC * NS
CHUNK = (B * L) // NW
BLK = 400
NBLK = CHUNK // BLK

_mesh = plsc.VectorSubcoreMesh(core_axis_name="c", subcore_axis_name="s")


@functools.partial(
    pl.kernel,
    out_type=jax.ShapeDtypeStruct((NW, CNT), jnp.float32),
    mesh=_mesh,
    compiler_params=pltpu.CompilerParams(needs_layout_passes=False),
    scratch_types=[
        pltpu.VMEM((CHUNK,), jnp.int32),
        pltpu.VMEM((CNT,), jnp.float32),
        pltpu.VMEM((BLK,), jnp.int32),
    ],
)
def _sc_hist(x_hbm, out_hbm, xbuf, cnt, ladd):
    wid = lax.axis_index("s") * NC + lax.axis_index("c")

    def zbody(j, _):
        cnt[pl.ds(j * 16, 16)] = jnp.zeros((16,), jnp.float32)
        return 0
    lax.fori_loop(0, CNT // 16, zbody, 0)

    for s in range(BLK // 16):
        t = lax.iota(jnp.int32, 16) + s * 16
        ladd[pl.ds(s * 16, 16)] = jnp.where(t >= L, t - L, t)

    pltpu.sync_copy(x_hbm.at[pl.ds(wid * CHUNK, CHUNK)], xbuf)

    ones = jnp.ones((16,), jnp.float32)

    def mbody(blk, _):
        base = blk * BLK
        for s in range(BLK // 16):
            v = xbuf[pl.ds(base + s * 16, 16)]
            idx = lax.shift_left(v, 8) + ladd[pl.ds(s * 16, 16)]
            plsc.addupdate_scatter(cnt, [idx], ones)
        return 0
    lax.fori_loop(0, NBLK, mbody, 0)

    pltpu.sync_copy(cnt, out_hbm.at[wid])


def _tc_body(cnt_ref, embed_ref, fcw_ref, bias_ref, out_ref):
    ct = jnp.sum(cnt_ref[...], axis=0)
    m = lax.dot_general(ct, embed_ref[...],
                        (((0,), (0,)), ((), ())),
                        preferred_element_type=jnp.float32)
    out = lax.dot_general(m * (1.0 / B), fcw_ref[...],
                          (((1,), (1,)), ((), ())),
                          preferred_element_type=jnp.float32)
    out_ref[...] = out[:L] + bias_ref[...]


def kernel(x, embed_weight, fc_weight, fc_bias):
    xflat = x.astype(jnp.int32).reshape(-1)
    counts = _sc_hist(xflat)
    cnt3 = counts.reshape(NW, V, LP)
    out = pl.pallas_call(
        _tc_body,
        out_shape=jax.ShapeDtypeStruct((L, V), jnp.float32),
    )(cnt3, embed_weight, fc_weight, fc_bias.reshape(1, V))
    return out


# speedup vs baseline: 166.4174x; 1.4835x over previous
import functools

import jax
import jax.numpy as jnp
from jax import lax
from jax.experimental import pallas as pl
from jax.experimental.pallas import tpu as pltpu
from jax.experimental.pallas import tpu_sc as plsc

B = 16384
L = 200
V = 17
D = 16
LP = 256
CNT = V * LP
NC, NS = 2, 16
NW = NC * NS
CHUNK = (B * L) // NW
BLK = 400
NBLK = CHUNK // BLK

_mesh = plsc.VectorSubcoreMesh(core_axis_name="c", subcore_axis_name="s")


@functools.partial(
    pl.kernel,
    out_type=jax.ShapeDtypeStruct((NW, CNT), jnp.float32),
    mesh=_mesh,
    compiler_params=pltpu.CompilerParams(needs_layout_passes=False),
    scratch_types=[
        pltpu.VMEM((CHUNK,), jnp.int32),
        pltpu.VMEM((CNT,), jnp.float32),
    ],
)
def _sc_hist(x_hbm, out_hbm, xbuf, cnt):
    wid = lax.axis_index("s") * NC + lax.axis_index("c")

    @plsc.parallel_loop(0, CNT // 16, unroll=4)
    def _(j):
        cnt[pl.ds(j * 16, 16)] = jnp.zeros((16,), jnp.float32)

    pltpu.sync_copy(x_hbm.at[pl.ds(wid * CHUNK, CHUNK)], xbuf)

    ones = jnp.ones((16,), jnp.float32)
    iota = lax.iota(jnp.int32, 16)
    lvecs = []
    for s in range(BLK // 16):
        t = iota + s * 16
        lvecs.append(jnp.where(t >= L, t - L, t))

    @plsc.parallel_loop(0, NBLK, unroll=2)
    def _(blk):
        base = blk * BLK
        for s in range(BLK // 16):
            v = xbuf[pl.ds(base + s * 16, 16)]
            idx = lax.shift_left(v, 8) + lvecs[s]
            plsc.addupdate_scatter(cnt, [idx], ones)

    pltpu.sync_copy(cnt, out_hbm.at[wid])


def _tc_body(cnt_ref, embed_ref, fcw_ref, bias_ref, out_ref):
    ct = jnp.sum(cnt_ref[...], axis=0)
    m = lax.dot_general(ct, embed_ref[...],
                        (((0,), (0,)), ((), ())),
                        preferred_element_type=jnp.float32)
    out = lax.dot_general(m * (1.0 / B), fcw_ref[...],
                          (((1,), (1,)), ((), ())),
                          preferred_element_type=jnp.float32)
    out_ref[...] = out[:L] + bias_ref[...]


def kernel(x, embed_weight, fc_weight, fc_bias):
    xflat = x.astype(jnp.int32).reshape(-1)
    counts = _sc_hist(xflat)
    cnt3 = counts.reshape(NW, V, LP)
    out = pl.pallas_call(
        _tc_body,
        out_shape=jax.ShapeDtypeStruct((L, V), jnp.float32),
    )(cnt3, embed_weight, fc_weight, fc_bias.reshape(1, V))
    return out


# speedup vs baseline: 262.0530x; 1.5747x over previous
import functools

import jax
import jax.numpy as jnp
from jax import lax
from jax.experimental import pallas as pl
from jax.experimental.pallas import tpu as pltpu
from jax.experimental.pallas import tpu_sc as plsc

B = 16384
L = 200
V = 17
D = 16
LP = 256
NC, NS = 2, 16
NW = NC * NS
ROWS = B // NW
CROWS = 128
NCHUNK = ROWS // CROWS
NSLICE = 13

_mesh = plsc.VectorSubcoreMesh(core_axis_name="c", subcore_axis_name="s")


@functools.partial(
    pl.kernel,
    out_type=jax.ShapeDtypeStruct((NW, V, LP), jnp.float32),
    mesh=_mesh,
    compiler_params=pltpu.CompilerParams(needs_layout_passes=False),
    scratch_types=[
        pltpu.VMEM((CROWS, L), jnp.int32),
        pltpu.VMEM((CROWS, L), jnp.int32),
        pltpu.VMEM((V, LP), jnp.float32),
        pltpu.SemaphoreType.DMA,
        pltpu.SemaphoreType.DMA,
    ],
)
def _sc_hist(x_hbm, out_hbm, xb0, xb1, cnt, sem0, sem1):
    wid = lax.axis_index("s") * NC + lax.axis_index("c")
    bufs = (xb0, xb1)
    sems = (sem0, sem1)

    @plsc.parallel_loop(0, V, unroll=1)
    def _(j):
        for s in range(LP // 16):
            cnt[j, pl.ds(s * 16, 16)] = jnp.zeros((16,), jnp.float32)

    row0 = wid * ROWS

    def start(k):
        return pltpu.async_copy(
            x_hbm.at[pl.ds(row0 + k * CROWS, CROWS)], bufs[k % 2], sems[k % 2])

    ones = jnp.ones((16,), jnp.float32)
    iota = lax.iota(jnp.int32, 16)
    tail_mask = iota >= 8
    lvecs = [iota + (c * 16 if c < NSLICE - 1 else L - 16)
             for c in range(NSLICE)]

    descs = [start(0), start(1), None, None]

    for k in range(NCHUNK):
        descs[k].wait()
        buf = bufs[k % 2]

        @plsc.parallel_loop(0, CROWS, unroll=2)
        def _(r):
            for c in range(NSLICE):
                off = c * 16 if c < NSLICE - 1 else L - 16
                v = buf[r, pl.ds(off, 16)]
                if c < NSLICE - 1:
                    plsc.addupdate_scatter(cnt, [v, lvecs[c]], ones)
                else:
                    plsc.addupdate_scatter(cnt, [v, lvecs[c]], ones,
                                           mask=tail_mask)

        if k + 2 < NCHUNK:
            descs[k + 2] = start(k + 2)

    pltpu.sync_copy(cnt, out_hbm.at[wid])


def _tc_body(cnt_ref, embed_ref, fcw_ref, bias_ref, out_ref):
    ct = jnp.sum(cnt_ref[...], axis=0)
    m = lax.dot_general(ct, embed_ref[...],
                        (((0,), (0,)), ((), ())),
                        preferred_element_type=jnp.float32)
    out = lax.dot_general(m * (1.0 / B), fcw_ref[...],
                          (((1,), (1,)), ((), ())),
                          preferred_element_type=jnp.float32)
    out_ref[...] = out[:L] + bias_ref[...]


def kernel(x, embed_weight, fc_weight, fc_bias):
    cnt3 = _sc_hist(x.astype(jnp.int32))
    out = pl.pallas_call(
        _tc_body,
        out_shape=jax.ShapeDtypeStruct((L, V), jnp.float32),
    )(cnt3, embed_weight, fc_weight, fc_bias.reshape(1, V))
    return out
